# Initial kernel scaffold; baseline (speedup 1.0000x reference)
import jax
import jax.numpy as jnp
from jax.experimental import pallas as pl


def kernel(x, expert_ids, expert_weights, gate_weights, up_weights, down_weights):
    raise NotImplementedError("write your pallas kernel here")


# speedup vs baseline: 1.0508x; 1.0508x over previous
import functools

import jax
import jax.numpy as jnp
from jax import lax
from jax.experimental import pallas as pl
from jax.experimental.pallas import tpu as pltpu
from jax.experimental.pallas import tpu_sc as plsc

_NC = 2
_NS = 16
_NW = _NC * _NS

_BM = 128


def _dispatch_meta(expert_ids, expert_weights, num_experts, cap, bm):
    nt, tk = expert_ids.shape
    p = nt * tk
    flat_e = expert_ids.reshape(-1).astype(jnp.int32)
    flat_w = expert_weights.reshape(-1)
    flat_t = (jnp.arange(p, dtype=jnp.int32) // tk).astype(jnp.int32)

    perm = jnp.argsort(flat_e)
    sorted_e = flat_e[perm]
    counts = jnp.bincount(flat_e, length=num_experts).astype(jnp.int32)
    start = jnp.concatenate(
        [jnp.zeros((1,), jnp.int32), jnp.cumsum(counts)[:-1]])
    padded = ((counts + bm - 1) // bm) * bm
    pad_end = jnp.cumsum(padded)
    pad_start = jnp.concatenate([jnp.zeros((1,), jnp.int32), pad_end[:-1]])

    rank = jnp.arange(p, dtype=jnp.int32) - start[sorted_e]
    slot = pad_start[sorted_e] + rank

    row_gather = jnp.zeros((cap,), jnp.int32).at[slot].set(flat_t[perm])
    row_weight = jnp.zeros((cap,), flat_w.dtype).at[slot].set(flat_w[perm])
    pos = jnp.zeros((p,), jnp.int32).at[perm].set(slot).reshape(nt, tk)

    nb = cap // bm
    block_rows = jnp.arange(nb, dtype=jnp.int32) * bm
    block_expert = jnp.minimum(
        jnp.searchsorted(pad_end, block_rows, side="right").astype(jnp.int32),
        num_experts - 1)
    return row_gather, row_weight, block_expert, pos


def _sc_gather(x, row_gather, cap):
    d = x.shape[1]
    per_w = cap // _NW
    chunk = 80 if per_w % 80 == 0 else per_w
    n_chunks = per_w // chunk
    mesh = plsc.VectorSubcoreMesh(core_axis_name="c", subcore_axis_name="s")

    @functools.partial(
        pl.kernel,
        mesh=mesh,
        out_type=jax.ShapeDtypeStruct((cap, d), x.dtype),
        scratch_types=[
            pltpu.VMEM((chunk,), jnp.int32),
            pltpu.VMEM((chunk, d), x.dtype),
            pltpu.SemaphoreType.DMA,
        ],
    )
    def gather_kernel(x_hbm, idx_hbm, out_hbm, idx_v, rows_v, sem):
        wid = lax.axis_index("s") * _NC + lax.axis_index("c")
        for c in range(n_chunks):
            base = wid * per_w + c * chunk
            pltpu.sync_copy(idx_hbm.at[pl.ds(base, chunk)], idx_v)
            pltpu.async_copy(x_hbm.at[idx_v], rows_v, sem).wait()
            pltpu.sync_copy(rows_v, out_hbm.at[pl.ds(base, chunk)])

    return gather_kernel(x, row_gather)


def _tc_act(xs, gate_w, up_w, block_expert, cap, bm):
    e, d, f = gate_w.shape
    bf = 1024
    nf = f // bf
    nb = cap // bm

    def act_kernel(be_ref, xs_ref, g_ref, u_ref, act_ref):
        xb = xs_ref[...]
        go = jnp.dot(xb, g_ref[0], preferred_element_type=jnp.float32)
        uo = jnp.dot(xb, u_ref[0], preferred_element_type=jnp.float32)
        act_ref[...] = (go * jax.nn.sigmoid(go)) * uo

    grid_spec = pltpu.PrefetchScalarGridSpec(
        num_scalar_prefetch=1,
        grid=(nf, nb),
        in_specs=[
            pl.BlockSpec((bm, d), lambda fi, b, be: (b, 0)),
            pl.BlockSpec((1, d, bf), lambda fi, b, be: (be[b], 0, fi)),
            pl.BlockSpec((1, d, bf), lambda fi, b, be: (be[b], 0, fi)),
        ],
        out_specs=pl.BlockSpec((bm, bf), lambda fi, b, be: (b, fi)),
    )
    return pl.pallas_call(
        act_kernel,
        grid_spec=grid_spec,
        out_shape=jax.ShapeDtypeStruct((cap, f), jnp.float32),
        compiler_params=pltpu.CompilerParams(
            dimension_semantics=("arbitrary", "arbitrary")),
    )(block_expert, xs, gate_w, up_w)


def _tc_down(act, down_w, row_weight, block_expert, cap, bm):
    e, f, d = down_w.shape
    nb = cap // bm
    w2d = row_weight.reshape(cap, 1)

    def down_kernel(be_ref, act_ref, d_ref, w_ref, out_ref):
        y = jnp.dot(act_ref[...], d_ref[0], preferred_element_type=jnp.float32)
        out_ref[...] = y * w_ref[...]

    grid_spec = pltpu.PrefetchScalarGridSpec(
        num_scalar_prefetch=1,
        grid=(nb,),
        in_specs=[
            pl.BlockSpec((bm, f), lambda b, be: (b, 0)),
            pl.BlockSpec((1, f, d), lambda b, be: (be[b], 0, 0)),
            pl.BlockSpec((bm, 1), lambda b, be: (b, 0)),
        ],
        out_specs=pl.BlockSpec((bm, d), lambda b, be: (b, 0)),
    )
    return pl.pallas_call(
        down_kernel,
        grid_spec=grid_spec,
        out_shape=jax.ShapeDtypeStruct((cap, d), jnp.float32),
        compiler_params=pltpu.CompilerParams(
            dimension_semantics=("arbitrary",)),
    )(block_expert, act, down_w, w2d)


def _sc_combine(y, pos, nt, d, tk):
    per_w = nt // _NW
    chunk = 32 if per_w % 32 == 0 else per_w
    n_chunks = per_w // chunk
    mesh = plsc.VectorSubcoreMesh(core_axis_name="c", subcore_axis_name="s")
    pos_cols = [pos[:, k].copy() for k in range(tk)]

    scratch = []
    for _ in range(tk):
        scratch.append(pltpu.VMEM((chunk,), jnp.int32))
        scratch.append(pltpu.VMEM((chunk, d), jnp.float32))
        scratch.append(pltpu.SemaphoreType.DMA)

    @functools.partial(
        pl.kernel,
        mesh=mesh,
        out_type=jax.ShapeDtypeStruct((nt, d), jnp.float32),
        scratch_types=scratch,
    )
    def combine_kernel(y_hbm, *rest):
        pos_hbm = rest[:tk]
        out_hbm = rest[tk]
        sc = rest[tk + 1:]
        idx_v = sc[0::3]
        buf_v = sc[1::3]
        sems = sc[2::3]
        wid = lax.axis_index("s") * _NC + lax.axis_index("c")
        for c in range(n_chunks):
            base = wid * per_w + c * chunk
            for k in range(tk):
                pltpu.sync_copy(pos_hbm[k].at[pl.ds(base, chunk)], idx_v[k])
            cps = [pltpu.async_copy(y_hbm.at[idx_v[k]], buf_v[k], sems[k])
                   for k in range(tk)]
            for cp in cps:
                cp.wait()

            def row_body(r, _):
                def col_body(ci, _):
                    off = ci * 64
                    for s in range(4):
                        acc = buf_v[0][r, pl.ds(off + s * 16, 16)]
                        for k in range(1, tk):
                            acc = acc + buf_v[k][r, pl.ds(off + s * 16, 16)]
                        buf_v[0][r, pl.ds(off + s * 16, 16)] = acc
                    return 0

                return lax.fori_loop(0, d // 64, col_body, 0)

            lax.fori_loop(0, chunk, row_body, 0)
            pltpu.sync_copy(buf_v[0], out_hbm.at[pl.ds(base, chunk)])

    return combine_kernel(y, *pos_cols)


def kernel(x, expert_ids, expert_weights, gate_weights, up_weights,
           down_weights):
    nt, d = x.shape
    tk = expert_ids.shape[1]
    num_experts = gate_weights.shape[0]
    p = nt * tk
    cap = p + num_experts * _BM

    row_gather, row_weight, block_expert, pos = _dispatch_meta(
        expert_ids, expert_weights, num_experts, cap, _BM)

    xs = _sc_gather(x, row_gather, cap)
    act = _tc_act(xs, gate_weights, up_weights, block_expert, cap, _BM)
    y = _tc_down(act, down_weights, row_weight, block_expert, cap, _BM)
    return _sc_combine(y, pos, nt, d, tk)


# speedup vs baseline: 1.0714x; 1.0196x over previous
import functools

import jax
import jax.numpy as jnp
from jax import lax
from jax.experimental import pallas as pl
from jax.experimental.pallas import tpu as pltpu
from jax.experimental.pallas import tpu_sc as plsc

_NC = 2
_NS = 16
_NW = _NC * _NS

_BM = 128


def _dispatch_meta(expert_ids, expert_weights, num_experts, cap, bm):
    nt, tk = expert_ids.shape
    p = nt * tk
    flat_e = expert_ids.reshape(-1).astype(jnp.int32)
    flat_w = expert_weights.reshape(-1)
    flat_t = (jnp.arange(p, dtype=jnp.int32) // tk).astype(jnp.int32)

    perm = jnp.argsort(flat_e)
    sorted_e = flat_e[perm]
    counts = jnp.bincount(flat_e, length=num_experts).astype(jnp.int32)
    start = jnp.concatenate(
        [jnp.zeros((1,), jnp.int32), jnp.cumsum(counts)[:-1]])
    padded = ((counts + bm - 1) // bm) * bm
    pad_end = jnp.cumsum(padded)
    pad_start = jnp.concatenate([jnp.zeros((1,), jnp.int32), pad_end[:-1]])

    rank = jnp.arange(p, dtype=jnp.int32) - start[sorted_e]
    slot = pad_start[sorted_e] + rank

    row_gather = jnp.zeros((cap,), jnp.int32).at[slot].set(flat_t[perm])
    row_weight = jnp.zeros((cap,), flat_w.dtype).at[slot].set(flat_w[perm])
    pos = jnp.zeros((p,), jnp.int32).at[perm].set(slot).reshape(nt, tk)

    nb = cap // bm
    block_rows = jnp.arange(nb, dtype=jnp.int32) * bm
    block_expert = jnp.minimum(
        jnp.searchsorted(pad_end, block_rows, side="right").astype(jnp.int32),
        num_experts - 1)
    return row_gather, row_weight, block_expert, pos


def _sc_gather(x, row_gather, cap):
    d = x.shape[1]
    per_w = cap // _NW
    chunk = 40 if per_w % 40 == 0 else per_w
    n_chunks = per_w // chunk
    nbuf = min(3, n_chunks)
    mesh = plsc.VectorSubcoreMesh(core_axis_name="c", subcore_axis_name="s")

    scratch = ([pltpu.VMEM((chunk,), jnp.int32) for _ in range(n_chunks)]
               + [pltpu.VMEM((chunk, d), x.dtype) for _ in range(nbuf)]
               + [pltpu.SemaphoreType.DMA for _ in range(2 * nbuf)])

    @functools.partial(
        pl.kernel,
        mesh=mesh,
        out_type=jax.ShapeDtypeStruct((cap, d), x.dtype),
        scratch_types=scratch,
    )
    def gather_kernel(x_hbm, idx_hbm, out_hbm, *sc):
        idx_v = sc[:n_chunks]
        rows_v = sc[n_chunks:n_chunks + nbuf]
        gsem = sc[n_chunks + nbuf:n_chunks + 2 * nbuf]
        wsem = sc[n_chunks + 2 * nbuf:]
        wid = lax.axis_index("s") * _NC + lax.axis_index("c")
        base = wid * per_w
        gq = [None] * n_chunks
        wq = [None] * n_chunks
        for c in range(n_chunks):
            b = c % nbuf
            if c >= nbuf:
                wq[c - nbuf].wait()
            pltpu.sync_copy(idx_hbm.at[pl.ds(base + c * chunk, chunk)],
                            idx_v[c])
            gq[c] = pltpu.async_copy(x_hbm.at[idx_v[c]], rows_v[b], gsem[b])
            if c >= 1:
                gq[c - 1].wait()
                wq[c - 1] = pltpu.async_copy(
                    rows_v[(c - 1) % nbuf],
                    out_hbm.at[pl.ds(base + (c - 1) * chunk, chunk)],
                    wsem[(c - 1) % nbuf])
        c = n_chunks - 1
        gq[c].wait()
        wq[c] = pltpu.async_copy(
            rows_v[c % nbuf], out_hbm.at[pl.ds(base + c * chunk, chunk)],
            wsem[c % nbuf])
        for c in range(max(0, n_chunks - nbuf), n_chunks):
            wq[c].wait()

    return gather_kernel(x, row_gather)


def _tc_act(xs, gate_w, up_w, block_expert, cap, bm):
    e, d, f = gate_w.shape
    bf = 1024
    nf = f // bf
    nb = cap // bm

    def act_kernel(be_ref, xs_ref, g_ref, u_ref, act_ref):
        xb = xs_ref[...].astype(jnp.bfloat16)
        g = g_ref[0].astype(jnp.bfloat16)
        u = u_ref[0].astype(jnp.bfloat16)
        go = jnp.dot(xb, g, preferred_element_type=jnp.float32)
        uo = jnp.dot(xb, u, preferred_element_type=jnp.float32)
        act_ref[...] = ((go * jax.nn.sigmoid(go)) * uo).astype(jnp.bfloat16)

    grid_spec = pltpu.PrefetchScalarGridSpec(
        num_scalar_prefetch=1,
        grid=(nf, nb),
        in_specs=[
            pl.BlockSpec((bm, d), lambda fi, b, be: (b, 0)),
            pl.BlockSpec((1, d, bf), lambda fi, b, be: (be[b], 0, fi)),
            pl.BlockSpec((1, d, bf), lambda fi, b, be: (be[b], 0, fi)),
        ],
        out_specs=pl.BlockSpec((bm, bf), lambda fi, b, be: (b, fi)),
    )
    return pl.pallas_call(
        act_kernel,
        grid_spec=grid_spec,
        out_shape=jax.ShapeDtypeStruct((cap, f), jnp.bfloat16),
        compiler_params=pltpu.CompilerParams(
            dimension_semantics=("arbitrary", "arbitrary")),
    )(block_expert, xs, gate_w, up_w)


def _tc_down(act, down_w, row_weight, block_expert, cap, bm):
    e, f, d = down_w.shape
    nb = cap // bm
    w2d = row_weight.reshape(cap, 1)

    def down_kernel(be_ref, act_ref, d_ref, w_ref, out_ref):
        dn = d_ref[0].astype(jnp.bfloat16)
        y = jnp.dot(act_ref[...], dn, preferred_element_type=jnp.float32)
        out_ref[...] = y * w_ref[...]

    grid_spec = pltpu.PrefetchScalarGridSpec(
        num_scalar_prefetch=1,
        grid=(nb,),
        in_specs=[
            pl.BlockSpec((bm, f), lambda b, be: (b, 0)),
            pl.BlockSpec((1, f, d), lambda b, be: (be[b], 0, 0)),
            pl.BlockSpec((bm, 1), lambda b, be: (b, 0)),
        ],
        out_specs=pl.BlockSpec((bm, d), lambda b, be: (b, 0)),
    )
    return pl.pallas_call(
        down_kernel,
        grid_spec=grid_spec,
        out_shape=jax.ShapeDtypeStruct((cap, d), jnp.float32),
        compiler_params=pltpu.CompilerParams(
            dimension_semantics=("arbitrary",)),
    )(block_expert, act, down_w, w2d)


def _sc_combine(y, pos, nt, d, tk):
    per_w = nt // _NW
    chunk = 32 if per_w % 32 == 0 else per_w
    n_chunks = per_w // chunk
    mesh = plsc.VectorSubcoreMesh(core_axis_name="c", subcore_axis_name="s")
    pos_cols = [pos[:, k].copy() for k in range(tk)]

    scratch = []
    for _ in range(tk):
        scratch.append(pltpu.VMEM((chunk,), jnp.int32))
        scratch.append(pltpu.VMEM((chunk, d), jnp.float32))
        scratch.append(pltpu.SemaphoreType.DMA)

    @functools.partial(
        pl.kernel,
        mesh=mesh,
        out_type=jax.ShapeDtypeStruct((nt, d), jnp.float32),
        scratch_types=scratch,
    )
    def combine_kernel(y_hbm, *rest):
        pos_hbm = rest[:tk]
        out_hbm = rest[tk]
        sc = rest[tk + 1:]
        idx_v = sc[0::3]
        buf_v = sc[1::3]
        sems = sc[2::3]
        wid = lax.axis_index("s") * _NC + lax.axis_index("c")
        for c in range(n_chunks):
            base = wid * per_w + c * chunk
            for k in range(tk):
                pltpu.sync_copy(pos_hbm[k].at[pl.ds(base, chunk)], idx_v[k])
            cps = [pltpu.async_copy(y_hbm.at[idx_v[k]], buf_v[k], sems[k])
                   for k in range(tk)]
            for cp in cps:
                cp.wait()

            def row_body(r, _):
                def col_body(ci, _):
                    off = ci * 64
                    for s in range(4):
                        acc = buf_v[0][r, pl.ds(off + s * 16, 16)]
                        for k in range(1, tk):
                            acc = acc + buf_v[k][r, pl.ds(off + s * 16, 16)]
                        buf_v[0][r, pl.ds(off + s * 16, 16)] = acc
                    return 0

                return lax.fori_loop(0, d // 64, col_body, 0)

            lax.fori_loop(0, chunk, row_body, 0)
            pltpu.sync_copy(buf_v[0], out_hbm.at[pl.ds(base, chunk)])

    return combine_kernel(y, *pos_cols)


def kernel(x, expert_ids, expert_weights, gate_weights, up_weights,
           down_weights):
    nt, d = x.shape
    tk = expert_ids.shape[1]
    num_experts = gate_weights.shape[0]
    p = nt * tk
    cap = p + num_experts * _BM

    row_gather, row_weight, block_expert, pos = _dispatch_meta(
        expert_ids, expert_weights, num_experts, cap, _BM)

    xs = _sc_gather(x, row_gather, cap)
    act = _tc_act(xs, gate_weights, up_weights, block_expert, cap, _BM)
    y = _tc_down(act, down_weights, row_weight, block_expert, cap, _BM)
    return _sc_combine(y, pos, nt, d, tk)


# speedup vs baseline: 1.2616x; 1.1776x over previous
import functools

import jax
import jax.numpy as jnp
from jax import lax
from jax.experimental import pallas as pl
from jax.experimental.pallas import tpu as pltpu
from jax.experimental.pallas import tpu_sc as plsc

_NC = 2
_NS = 16
_NW = _NC * _NS

_BM = 128


def _dispatch_meta(expert_ids, expert_weights, num_experts, cap, bm):
    nt, tk = expert_ids.shape
    p = nt * tk
    flat_e = expert_ids.reshape(-1).astype(jnp.int32)
    flat_w = expert_weights.reshape(-1)
    flat_t = (jnp.arange(p, dtype=jnp.int32) // tk).astype(jnp.int32)

    perm = jnp.argsort(flat_e)
    sorted_e = flat_e[perm]
    counts = jnp.bincount(flat_e, length=num_experts).astype(jnp.int32)
    start = jnp.concatenate(
        [jnp.zeros((1,), jnp.int32), jnp.cumsum(counts)[:-1]])
    padded = ((counts + bm - 1) // bm) * bm
    pad_end = jnp.cumsum(padded)
    pad_start = jnp.concatenate([jnp.zeros((1,), jnp.int32), pad_end[:-1]])

    rank = jnp.arange(p, dtype=jnp.int32) - start[sorted_e]
    slot = pad_start[sorted_e] + rank

    row_gather = jnp.zeros((cap,), jnp.int32).at[slot].set(flat_t[perm])
    row_weight = jnp.zeros((cap,), flat_w.dtype).at[slot].set(flat_w[perm])
    pos = jnp.zeros((p,), jnp.int32).at[perm].set(slot).reshape(nt, tk)

    nb = cap // bm
    block_rows = jnp.arange(nb, dtype=jnp.int32) * bm
    block_expert = jnp.minimum(
        jnp.searchsorted(pad_end, block_rows, side="right").astype(jnp.int32),
        num_experts - 1)
    return row_gather, row_weight, block_expert, pos


def _sc_gather(x, row_gather, cap):
    d = x.shape[1]
    per_w = cap // _NW
    chunk = 32 if per_w % 32 == 0 else per_w
    n_chunks = per_w // chunk
    nbuf = min(3, n_chunks)
    mesh = plsc.VectorSubcoreMesh(core_axis_name="c", subcore_axis_name="s")

    scratch = ([pltpu.VMEM((per_w,), jnp.int32)]
               + [pltpu.VMEM((chunk, d), x.dtype) for _ in range(nbuf)]
               + [pltpu.SemaphoreType.DMA for _ in range(2 * nbuf)])

    @functools.partial(
        pl.kernel,
        mesh=mesh,
        out_type=jax.ShapeDtypeStruct((cap, d), x.dtype),
        scratch_types=scratch,
    )
    def gather_kernel(x_hbm, idx_hbm, out_hbm, idx_v, *sc):
        rows_v = sc[:nbuf]
        gsem = sc[nbuf:2 * nbuf]
        wsem = sc[2 * nbuf:]
        wid = lax.axis_index("s") * _NC + lax.axis_index("c")
        base = wid * per_w
        pltpu.sync_copy(idx_hbm.at[pl.ds(base, per_w)], idx_v)
        gq = [None] * n_chunks
        wq = [None] * n_chunks
        for c in range(n_chunks):
            b = c % nbuf
            if c >= nbuf:
                wq[c - nbuf].wait()
            gq[c] = pltpu.async_copy(
                x_hbm.at[idx_v.at[pl.ds(c * chunk, chunk)]], rows_v[b],
                gsem[b])
            if c >= nbuf - 1:
                j = c - (nbuf - 1)
                gq[j].wait()
                wq[j] = pltpu.async_copy(
                    rows_v[j % nbuf],
                    out_hbm.at[pl.ds(base + j * chunk, chunk)],
                    wsem[j % nbuf])
        for j in range(max(0, n_chunks - (nbuf - 1)), n_chunks):
            gq[j].wait()
            wq[j] = pltpu.async_copy(
                rows_v[j % nbuf],
                out_hbm.at[pl.ds(base + j * chunk, chunk)],
                wsem[j % nbuf])
        for j in range(max(0, n_chunks - nbuf), n_chunks):
            wq[j].wait()

    return gather_kernel(x, row_gather)


def _tc_moe_mlp(xs, gate_w, up_w, down_w, row_weight, block_expert, cap, bm):
    e, d, f = gate_w.shape
    nb = cap // bm
    w2d = row_weight.reshape(cap, 1)

    def mlp_kernel(be_ref, xs_ref, g_ref, u_ref, d_ref, w_ref, out_ref):
        xb = xs_ref[...]
        go = jnp.dot(xb, g_ref[0], preferred_element_type=jnp.float32)
        uo = jnp.dot(xb, u_ref[0], preferred_element_type=jnp.float32)
        act = (go * jax.nn.sigmoid(go)) * uo
        y = jnp.dot(act, d_ref[0], preferred_element_type=jnp.float32)
        out_ref[...] = y * w_ref[...]

    grid_spec = pltpu.PrefetchScalarGridSpec(
        num_scalar_prefetch=1,
        grid=(nb,),
        in_specs=[
            pl.BlockSpec((bm, d), lambda b, be: (b, 0)),
            pl.BlockSpec((1, d, f), lambda b, be: (be[b], 0, 0)),
            pl.BlockSpec((1, d, f), lambda b, be: (be[b], 0, 0)),
            pl.BlockSpec((1, f, d), lambda b, be: (be[b], 0, 0)),
            pl.BlockSpec((bm, 1), lambda b, be: (b, 0)),
        ],
        out_specs=pl.BlockSpec((bm, d), lambda b, be: (b, 0)),
    )
    return pl.pallas_call(
        mlp_kernel,
        grid_spec=grid_spec,
        out_shape=jax.ShapeDtypeStruct((cap, d), jnp.float32),
        compiler_params=pltpu.CompilerParams(
            dimension_semantics=("arbitrary",)),
    )(block_expert, xs, gate_w, up_w, down_w, w2d)


def _sc_combine(y, pos, nt, d, tk):
    per_w = nt // _NW
    chunk = 32 if per_w % 32 == 0 else per_w
    n_chunks = per_w // chunk
    mesh = plsc.VectorSubcoreMesh(core_axis_name="c", subcore_axis_name="s")
    pos_cols = [pos[:, k].copy() for k in range(tk)]

    scratch = []
    for _ in range(tk):
        scratch.append(pltpu.VMEM((chunk,), jnp.int32))
        scratch.append(pltpu.VMEM((chunk, d), jnp.float32))
        scratch.append(pltpu.SemaphoreType.DMA)

    @functools.partial(
        pl.kernel,
        mesh=mesh,
        out_type=jax.ShapeDtypeStruct((nt, d), jnp.float32),
        scratch_types=scratch,
    )
    def combine_kernel(y_hbm, *rest):
        pos_hbm = rest[:tk]
        out_hbm = rest[tk]
        sc = rest[tk + 1:]
        idx_v = sc[0::3]
        buf_v = sc[1::3]
        sems = sc[2::3]
        wid = lax.axis_index("s") * _NC + lax.axis_index("c")
        for c in range(n_chunks):
            base = wid * per_w + c * chunk
            for k in range(tk):
                pltpu.sync_copy(pos_hbm[k].at[pl.ds(base, chunk)], idx_v[k])
            cps = [pltpu.async_copy(y_hbm.at[idx_v[k]], buf_v[k], sems[k])
                   for k in range(tk)]
            for cp in cps:
                cp.wait()

            def row_body(r, _):
                def col_body(ci, _):
                    off = ci * 64
                    for s in range(4):
                        acc = buf_v[0][r, pl.ds(off + s * 16, 16)]
                        for k in range(1, tk):
                            acc = acc + buf_v[k][r, pl.ds(off + s * 16, 16)]
                        buf_v[0][r, pl.ds(off + s * 16, 16)] = acc
                    return 0

                return lax.fori_loop(0, d // 64, col_body, 0)

            lax.fori_loop(0, chunk, row_body, 0)
            pltpu.sync_copy(buf_v[0], out_hbm.at[pl.ds(base, chunk)])

    return combine_kernel(y, *pos_cols)


def kernel(x, expert_ids, expert_weights, gate_weights, up_weights,
           down_weights):
    nt, d = x.shape
    tk = expert_ids.shape[1]
    num_experts = gate_weights.shape[0]
    p = nt * tk
    cap = p + num_experts * _BM

    row_gather, row_weight, block_expert, pos = _dispatch_meta(
        expert_ids, expert_weights, num_experts, cap, _BM)

    xs = _sc_gather(x, row_gather, cap)
    y = _tc_moe_mlp(xs, gate_weights, up_weights, down_weights, row_weight,
                    block_expert, cap, _BM)
    return _sc_combine(y, pos, nt, d, tk)


# speedup vs baseline: 1.8473x; 1.4643x over previous
import functools

import jax
import jax.numpy as jnp
from jax import lax
from jax.experimental import pallas as pl
from jax.experimental.pallas import tpu as pltpu
from jax.experimental.pallas import tpu_sc as plsc

_NC = 2
_NS = 16
_NW = _NC * _NS

_BM = 128


def _dispatch_meta(expert_ids, num_experts, cap, bm):
    nt, tk = expert_ids.shape
    p = nt * tk
    flat_e = expert_ids.reshape(-1).astype(jnp.int32)

    onehot = (flat_e[:, None] == jnp.arange(num_experts, dtype=jnp.int32)
              ).astype(jnp.int32)
    cum = jnp.cumsum(onehot, axis=0)
    rank = jnp.take_along_axis(cum, flat_e[:, None], axis=1)[:, 0] - 1
    counts = cum[-1]

    padded = ((counts + bm - 1) // bm) * bm
    pad_end = jnp.cumsum(padded)
    pad_start = pad_end - padded

    slot = pad_start[flat_e] + rank

    nb = cap // bm
    block_rows = jnp.arange(nb, dtype=jnp.int32)[:, None] * bm
    block_expert = jnp.minimum(
        jnp.sum((block_rows >= pad_end[None, :]).astype(jnp.int32), axis=1),
        num_experts - 1).astype(jnp.int32)
    return slot, block_expert


def _sc_dispatch(x, slot_3d, cap, tk):
    nt, d = x.shape
    per_w = nt // _NW
    mesh = plsc.VectorSubcoreMesh(core_axis_name="c", subcore_axis_name="s")

    scratch = ([pltpu.VMEM((per_w, d), x.dtype),
                pltpu.VMEM((tk, per_w), jnp.int32)]
               + [pltpu.SemaphoreType.DMA for _ in range(tk)])

    @functools.partial(
        pl.kernel,
        mesh=mesh,
        out_type=jax.ShapeDtypeStruct((cap, d), x.dtype),
        scratch_types=scratch,
    )
    def dispatch_kernel(x_hbm, idx_hbm, out_hbm, xbuf, idx_v, *sems):
        wid = lax.axis_index("s") * _NC + lax.axis_index("c")
        base = wid * per_w
        pltpu.sync_copy(idx_hbm.at[wid], idx_v)
        pltpu.sync_copy(x_hbm.at[pl.ds(base, per_w)], xbuf)
        cps = [pltpu.async_copy(xbuf, out_hbm.at[idx_v.at[k]], sems[k])
               for k in range(tk)]
        for cp in cps:
            cp.wait()

    return dispatch_kernel(x, slot_3d)


def _tc_moe_mlp(xs, gate_w, up_w, down_w, block_expert, cap, bm):
    e, d, f = gate_w.shape
    nb = cap // bm

    def mlp_kernel(be_ref, xs_ref, g_ref, u_ref, d_ref, out_ref):
        xb = xs_ref[...]
        go = jnp.dot(xb, g_ref[0], preferred_element_type=jnp.float32)
        uo = jnp.dot(xb, u_ref[0], preferred_element_type=jnp.float32)
        act = (go * jax.nn.sigmoid(go)) * uo
        out_ref[...] = jnp.dot(act, d_ref[0],
                               preferred_element_type=jnp.float32)

    grid_spec = pltpu.PrefetchScalarGridSpec(
        num_scalar_prefetch=1,
        grid=(nb,),
        in_specs=[
            pl.BlockSpec((bm, d), lambda b, be: (b, 0)),
            pl.BlockSpec((1, d, f), lambda b, be: (be[b], 0, 0)),
            pl.BlockSpec((1, d, f), lambda b, be: (be[b], 0, 0)),
            pl.BlockSpec((1, f, d), lambda b, be: (be[b], 0, 0)),
        ],
        out_specs=pl.BlockSpec((bm, d), lambda b, be: (b, 0)),
    )
    return pl.pallas_call(
        mlp_kernel,
        grid_spec=grid_spec,
        out_shape=jax.ShapeDtypeStruct((cap, d), jnp.float32),
        compiler_params=pltpu.CompilerParams(
            dimension_semantics=("arbitrary",)),
    )(block_expert, xs, gate_w, up_w, down_w)


def _sc_combine(y, pos_cols, w_cols, nt, d, tk):
    per_w = nt // _NW
    chunk = 32 if per_w % 32 == 0 else per_w
    n_chunks = per_w // chunk
    mesh = plsc.VectorSubcoreMesh(core_axis_name="c", subcore_axis_name="s")

    scratch = []
    for _ in range(tk):
        scratch.append(pltpu.VMEM((chunk,), jnp.int32))
        scratch.append(pltpu.VMEM((chunk, 16), jnp.float32))
        scratch.append(pltpu.VMEM((chunk, d), jnp.float32))
        scratch.append(pltpu.SemaphoreType.DMA)

    @functools.partial(
        pl.kernel,
        mesh=mesh,
        out_type=jax.ShapeDtypeStruct((nt, d), jnp.float32),
        scratch_types=scratch,
    )
    def combine_kernel(y_hbm, *rest):
        pos_hbm = rest[:tk]
        w_hbm = rest[tk:2 * tk]
        out_hbm = rest[2 * tk]
        sc = rest[2 * tk + 1:]
        idx_v = sc[0::4]
        w_v = sc[1::4]
        buf_v = sc[2::4]
        sems = sc[3::4]
        wid = lax.axis_index("s") * _NC + lax.axis_index("c")
        for c in range(n_chunks):
            base = wid * per_w + c * chunk
            for k in range(tk):
                pltpu.sync_copy(pos_hbm[k].at[pl.ds(base, chunk)], idx_v[k])
                pltpu.sync_copy(w_hbm[k].at[pl.ds(base, chunk)], w_v[k])
            cps = [pltpu.async_copy(y_hbm.at[idx_v[k]], buf_v[k], sems[k])
                   for k in range(tk)]
            for cp in cps:
                cp.wait()

            def row_body(r, _):
                wk = [w_v[k][r, :] for k in range(tk)]

                def col_body(ci, _):
                    off = ci * 64
                    for s in range(4):
                        sl = pl.ds(off + s * 16, 16)
                        acc = buf_v[0][r, sl] * wk[0]
                        for k in range(1, tk):
                            acc = acc + buf_v[k][r, sl] * wk[k]
                        buf_v[0][r, sl] = acc
                    return 0

                return lax.fori_loop(0, d // 64, col_body, 0)

            lax.fori_loop(0, chunk, row_body, 0)
            pltpu.sync_copy(buf_v[0], out_hbm.at[pl.ds(base, chunk)])

    return combine_kernel(y, *pos_cols, *w_cols)


def kernel(x, expert_ids, expert_weights, gate_weights, up_weights,
           down_weights):
    nt, d = x.shape
    tk = expert_ids.shape[1]
    num_experts = gate_weights.shape[0]
    p = nt * tk
    cap = p + num_experts * _BM

    slot, block_expert = _dispatch_meta(expert_ids, num_experts, cap, _BM)

    per_w = nt // _NW
    slot_3d = slot.reshape(_NW, per_w, tk).transpose(0, 2, 1).copy()
    pos2d = slot.reshape(nt, tk)
    pos_cols = [pos2d[:, k].copy() for k in range(tk)]
    w_cols = [jnp.broadcast_to(expert_weights[:, k][:, None], (nt, 16)).copy()
              for k in range(tk)]

    xs = _sc_dispatch(x, slot_3d, cap, tk)
    y = _tc_moe_mlp(xs, gate_weights, up_weights, down_weights, block_expert,
                    cap, _BM)
    return _sc_combine(y, pos_cols, w_cols, nt, d, tk)


# speedup vs baseline: 1.9013x; 1.0292x over previous
import functools

import jax
import jax.numpy as jnp
from jax import lax
from jax.experimental import pallas as pl
from jax.experimental.pallas import tpu as pltpu
from jax.experimental.pallas import tpu_sc as plsc

_NC = 2
_NS = 16
_NW = _NC * _NS

_BM = 128


def _dispatch_meta(expert_ids, num_experts, cap, bm):
    nt, tk = expert_ids.shape
    p = nt * tk
    flat_e = expert_ids.reshape(-1).astype(jnp.int32)

    onehot = (flat_e[:, None] == jnp.arange(num_experts, dtype=jnp.int32)
              ).astype(jnp.int32)
    cum = jnp.cumsum(onehot, axis=0)
    rank = jnp.take_along_axis(cum, flat_e[:, None], axis=1)[:, 0] - 1
    counts = cum[-1]

    padded = ((counts + bm - 1) // bm) * bm
    pad_end = jnp.cumsum(padded)
    pad_start = pad_end - padded

    slot = pad_start[flat_e] + rank

    nb = cap // bm
    block_rows = jnp.arange(nb, dtype=jnp.int32)[:, None] * bm
    block_expert = jnp.minimum(
        jnp.sum((block_rows >= pad_end[None, :]).astype(jnp.int32), axis=1),
        num_experts - 1).astype(jnp.int32)
    return slot, block_expert


def _sc_dispatch(x, slot_3d, cap, tk):
    nt, d = x.shape
    per_w = nt // _NW
    mesh = plsc.VectorSubcoreMesh(core_axis_name="c", subcore_axis_name="s")

    scratch = ([pltpu.VMEM((per_w, d), x.dtype),
                pltpu.VMEM((tk, per_w), jnp.int32)]
               + [pltpu.SemaphoreType.DMA for _ in range(tk + 1)])

    @functools.partial(
        pl.kernel,
        mesh=mesh,
        out_type=jax.ShapeDtypeStruct((cap, d), x.dtype),
        scratch_types=scratch,
    )
    def dispatch_kernel(x_hbm, idx_hbm, out_hbm, xbuf, idx_v, *sems):
        wid = lax.axis_index("s") * _NC + lax.axis_index("c")
        base = wid * per_w
        xcp = pltpu.async_copy(x_hbm.at[pl.ds(base, per_w)], xbuf, sems[tk])
        pltpu.sync_copy(idx_hbm.at[wid], idx_v)
        xcp.wait()
        cps = [pltpu.async_copy(xbuf, out_hbm.at[idx_v.at[k]], sems[k])
               for k in range(tk)]
        for cp in cps:
            cp.wait()

    return dispatch_kernel(x, slot_3d)


def _tc_moe_mlp(xs, gate_w, up_w, down_w, block_expert, cap, bm):
    e, d, f = gate_w.shape
    nb = cap // bm

    def mlp_kernel(be_ref, xs_ref, g_ref, u_ref, d_ref, out_ref):
        xb = xs_ref[...]
        go = jnp.dot(xb, g_ref[0], preferred_element_type=jnp.float32)
        uo = jnp.dot(xb, u_ref[0], preferred_element_type=jnp.float32)
        act = (go * jax.nn.sigmoid(go)) * uo
        out_ref[...] = jnp.dot(act, d_ref[0],
                               preferred_element_type=jnp.float32)

    grid_spec = pltpu.PrefetchScalarGridSpec(
        num_scalar_prefetch=1,
        grid=(nb,),
        in_specs=[
            pl.BlockSpec((bm, d), lambda b, be: (b, 0)),
            pl.BlockSpec((1, d, f), lambda b, be: (be[b], 0, 0)),
            pl.BlockSpec((1, d, f), lambda b, be: (be[b], 0, 0)),
            pl.BlockSpec((1, f, d), lambda b, be: (be[b], 0, 0)),
        ],
        out_specs=pl.BlockSpec((bm, d), lambda b, be: (b, 0)),
    )
    return pl.pallas_call(
        mlp_kernel,
        grid_spec=grid_spec,
        out_shape=jax.ShapeDtypeStruct((cap, d), jnp.float32),
        compiler_params=pltpu.CompilerParams(
            dimension_semantics=("arbitrary",)),
    )(block_expert, xs, gate_w, up_w, down_w)


def _sc_combine(y, pos_cols, w_cols, nt, d, tk):
    per_w = nt // _NW
    chunk = 16 if per_w % 16 == 0 else per_w
    n_chunks = per_w // chunk
    mesh = plsc.VectorSubcoreMesh(core_axis_name="c", subcore_axis_name="s")

    scratch = []
    for _ in range(tk):
        scratch.append(pltpu.VMEM((per_w,), jnp.int32))
        scratch.append(pltpu.VMEM((per_w, 16), jnp.float32))
        scratch.append(pltpu.VMEM((2, chunk, d), jnp.float32))
        scratch.append(pltpu.SemaphoreType.DMA)
        scratch.append(pltpu.SemaphoreType.DMA)
    scratch.append(pltpu.SemaphoreType.DMA)
    scratch.append(pltpu.SemaphoreType.DMA)

    @functools.partial(
        pl.kernel,
        mesh=mesh,
        out_type=jax.ShapeDtypeStruct((nt, d), jnp.float32),
        scratch_types=scratch,
    )
    def combine_kernel(y_hbm, *rest):
        pos_hbm = rest[:tk]
        w_hbm = rest[tk:2 * tk]
        out_hbm = rest[2 * tk]
        sc = rest[2 * tk + 1:]
        idx_v = sc[0:5 * tk:5]
        w_v = sc[1:5 * tk:5]
        buf_v = sc[2:5 * tk:5]
        gsem = [sc[5 * k + 3:5 * k + 5] for k in range(tk)]
        wsem = sc[5 * tk:]
        wid = lax.axis_index("s") * _NC + lax.axis_index("c")
        base = wid * per_w
        for k in range(tk):
            pltpu.sync_copy(pos_hbm[k].at[pl.ds(base, per_w)], idx_v[k])
            pltpu.sync_copy(w_hbm[k].at[pl.ds(base, per_w)], w_v[k])

        def start_wave(c):
            par = c % 2
            return [pltpu.async_copy(
                y_hbm.at[idx_v[k].at[pl.ds(c * chunk, chunk)]],
                buf_v[k].at[par], gsem[k][par]) for k in range(tk)]

        def add_wave(c):
            par = c % 2

            def row_body(r, _):
                wk = [w_v[k][c * chunk + r, :] for k in range(tk)]

                def col_body(ci, _):
                    off = ci * 64
                    for s in range(4):
                        sl = pl.ds(off + s * 16, 16)
                        acc = buf_v[0][par, r, sl] * wk[0]
                        for k in range(1, tk):
                            acc = acc + buf_v[k][par, r, sl] * wk[k]
                        buf_v[0][par, r, sl] = acc
                    return 0

                return lax.fori_loop(0, d // 64, col_body, 0)

            lax.fori_loop(0, chunk, row_body, 0)
            return pltpu.async_copy(
                buf_v[0].at[par], out_hbm.at[pl.ds(base + c * chunk, chunk)],
                wsem[par])

        gq = [None] * n_chunks
        wq = [None] * n_chunks
        gq[0] = start_wave(0)
        for c in range(n_chunks):
            if c + 1 < n_chunks:
                if c >= 1:
                    wq[c - 1].wait()
                gq[c + 1] = start_wave(c + 1)
            for cp in gq[c]:
                cp.wait()
            wq[c] = add_wave(c)
        for c in range(max(0, n_chunks - 2), n_chunks):
            wq[c].wait()

    return combine_kernel(y, *pos_cols, *w_cols)


def kernel(x, expert_ids, expert_weights, gate_weights, up_weights,
           down_weights):
    nt, d = x.shape
    tk = expert_ids.shape[1]
    num_experts = gate_weights.shape[0]
    p = nt * tk
    cap = p + num_experts * _BM

    slot, block_expert = _dispatch_meta(expert_ids, num_experts, cap, _BM)

    per_w = nt // _NW
    slot_3d = slot.reshape(_NW, per_w, tk).transpose(0, 2, 1).copy()
    pos2d = slot.reshape(nt, tk)
    pos_cols = [pos2d[:, k].copy() for k in range(tk)]
    w_cols = [jnp.broadcast_to(expert_weights[:, k][:, None], (nt, 16)).copy()
              for k in range(tk)]

    xs = _sc_dispatch(x, slot_3d, cap, tk)
    y = _tc_moe_mlp(xs, gate_weights, up_weights, down_weights, block_expert,
                    cap, _BM)
    return _sc_combine(y, pos_cols, w_cols, nt, d, tk)
